# baseline (speedup 1.0000x reference)
import functools
import math

import jax
import jax.numpy as jnp
from jax import lax
from jax.experimental import pallas as pl
from jax.experimental.pallas import tpu as pltpu
from jax.experimental.pallas import tpu_sc as plsc

_N = 20000
_NCLS = 91
_NREG = 364
_LANES = 128
_NW = 32
_SUBCAP = 144
_CAP = 16 * _SUBCAP
_C = _NW * _CAP
_ROWS = _C // 128
_THR = 0.05
_NMS_THR = 0.5
_DETS = 100
_CLIP = math.log(1000.0 / 16)
_RPT = _N // _NW
_SLAB = _RPT + 7
_CH = 80
_NCHUNK = 8


def _prep_body(hw_ref, x_ref, reg_ref, prop_ref,
               ms_ref, x1_ref, y1_ref, x2_ref, y2_ref):
    wf = hw_ref[0]
    hf = hw_ref[1]
    x = x_ref[...]
    mx = jnp.max(x, axis=1, keepdims=True)
    e = jnp.exp(x - mx)
    s = e / jnp.sum(e, axis=1, keepdims=True)

    reg = reg_ref[...]
    i0 = lax.broadcasted_iota(jnp.int32, (_NREG, _LANES), 0)
    i1 = lax.broadcasted_iota(jnp.int32, (_NREG, _LANES), 1)
    s0 = (i0 == i1 * 4).astype(jnp.float32)
    s1 = (i0 == i1 * 4 + 1).astype(jnp.float32)
    s2 = (i0 == i1 * 4 + 2).astype(jnp.float32)
    s3 = (i0 == i1 * 4 + 3).astype(jnp.float32)
    f32 = jnp.float32
    dx = jnp.dot(reg, s0, preferred_element_type=f32) / 10.0
    dy = jnp.dot(reg, s1, preferred_element_type=f32) / 10.0
    dw = jnp.minimum(jnp.dot(reg, s2, preferred_element_type=f32) / 5.0, _CLIP)
    dh = jnp.minimum(jnp.dot(reg, s3, preferred_element_type=f32) / 5.0, _CLIP)

    p = prop_ref[...]
    w = (p[:, 2] - p[:, 0])[:, None]
    h = (p[:, 3] - p[:, 1])[:, None]
    cx = p[:, 0][:, None] + 0.5 * w
    cy = p[:, 1][:, None] + 0.5 * h
    pcx = dx * w + cx
    pcy = dy * h + cy
    pw = jnp.exp(dw) * w
    ph = jnp.exp(dh) * h
    x1 = jnp.clip(pcx - 0.5 * pw, 0.0, wf)
    y1 = jnp.clip(pcy - 0.5 * ph, 0.0, hf)
    x2 = jnp.clip(pcx + 0.5 * pw, 0.0, wf)
    y2 = jnp.clip(pcy + 0.5 * ph, 0.0, hf)

    lane = lax.broadcasted_iota(jnp.int32, x.shape, 1)
    ok = ((lane >= 1) & (lane < _NCLS) & (s > _THR)
          & ((x2 - x1) >= 0.01) & ((y2 - y1) >= 0.01))
    ms_ref[...] = jnp.where(ok, s, 0.0)
    x1_ref[...] = x1
    y1_ref[...] = y1
    x2_ref[...] = x2
    y2_ref[...] = y2


def _prep(hw, logits_padded, box_regression, proposals):
    nb = 10
    br = _N // nb
    out = jax.ShapeDtypeStruct((_N, _LANES), jnp.float32)
    return pl.pallas_call(
        _prep_body,
        grid=(nb,),
        in_specs=[
            pl.BlockSpec(memory_space=pltpu.SMEM),
            pl.BlockSpec((br, _LANES), lambda i: (i, 0)),
            pl.BlockSpec((br, _NREG), lambda i: (i, 0)),
            pl.BlockSpec((br, 4), lambda i: (i, 0)),
        ],
        out_specs=[pl.BlockSpec((br, _LANES), lambda i: (i, 0))] * 5,
        out_shape=(out,) * 5,
    )(hw, logits_padded, box_regression, proposals)


def _sc_body(ms_hbm, x1_hbm, y1_hbm, x2_hbm, y2_hbm,
             clbl_hbm, cs_hbm, cx1_hbm, cy1_hbm, cx2_hbm, cy2_hbm,
             mb0, mb1, xb0, xb1, yb0, yb1, zb0, zb1, wb0, wb1,
             cs, cl, cx1, cy1, cx2, cy2,
             sm, sx, sy, sz, sw):
    wid = lax.axis_index("s") * 2 + lax.axis_index("c")
    rst = pl.multiple_of((wid * _RPT) & -8, 8)
    rlo = wid * _RPT - rst

    iota16 = lax.iota(jnp.int32, 16)
    z16f = jnp.zeros((16,), jnp.float32)

    def zero_body(i, carry):
        cs[pl.ds(i * 16, 16)] = z16f
        return carry

    lax.fori_loop(0, _CAP // 16, zero_body, 0)

    mbufs = (mb0, mb1)
    xbufs = (xb0, xb1)
    ybufs = (yb0, yb1)
    zbufs = (zb0, zb1)
    wbufs = (wb0, wb1)

    def fire(ch):
        b = ch % 2
        r0 = ch * _CH
        nr = _SLAB - r0 if r0 + _CH > _SLAB else _CH
        sl = pl.ds(rst + r0, nr)
        dst = pl.ds(0, nr)
        return (
            pltpu.async_copy(ms_hbm.at[sl], mbufs[b].at[dst], sm),
            pltpu.async_copy(x1_hbm.at[sl], xbufs[b].at[dst], sx),
            pltpu.async_copy(y1_hbm.at[sl], ybufs[b].at[dst], sy),
            pltpu.async_copy(x2_hbm.at[sl], zbufs[b].at[dst], sz),
            pltpu.async_copy(y2_hbm.at[sl], wbufs[b].at[dst], sw),
        )

    cps = fire(0)
    off_v = iota16
    for ch in range(_NCHUNK):
        nxt = fire(ch + 1) if ch + 1 < _NCHUNK else None
        for cp in cps:
            cp.wait()
        cps = nxt
        b = ch % 2
        mbuf = mbufs[b]
        xbuf = xbufs[b]
        ybuf = ybufs[b]
        zbuf = zbufs[b]
        wbuf = wbufs[b]
        r0 = ch * _CH
        nr = _SLAB - r0 if r0 + _CH > _SLAB else _CH
        lo = jnp.maximum(rlo, r0)
        hi = jnp.minimum(rlo + _RPT, r0 + nr)

        def scan_row(lrow, off_v, r0=r0, mbuf=mbuf, xbuf=xbuf, ybuf=ybuf,
                     zbuf=zbuf, wbuf=wbuf):
            crow = lrow - r0
            for lg in range(_LANES // 16):
                slg = pl.ds(lg * 16, 16)
                s = mbuf[crow, slg]
                ok = (s > _THR) & (off_v < _CAP)
                plsc.store_scatter(cs, [off_v], s, mask=ok)
                plsc.store_scatter(cl, [off_v], lg * 16 + iota16, mask=ok)
                plsc.store_scatter(cx1, [off_v], xbuf[crow, slg], mask=ok)
                plsc.store_scatter(cy1, [off_v], ybuf[crow, slg], mask=ok)
                plsc.store_scatter(cx2, [off_v], zbuf[crow, slg], mask=ok)
                plsc.store_scatter(cy2, [off_v], wbuf[crow, slg], mask=ok)
                off_v = off_v + jnp.where(ok, 16, 0)
            return off_v

        off_v = lax.fori_loop(lo, hi, scan_row, off_v)

    pltpu.sync_copy(cl, clbl_hbm.at[wid])
    pltpu.sync_copy(cs, cs_hbm.at[wid])
    pltpu.sync_copy(cx1, cx1_hbm.at[wid])
    pltpu.sync_copy(cy1, cy1_hbm.at[wid])
    pltpu.sync_copy(cx2, cx2_hbm.at[wid])
    pltpu.sync_copy(cy2, cy2_hbm.at[wid])


@functools.cache
def _get_sc_kernel():
    mesh = plsc.VectorSubcoreMesh(
        core_axis_name="c", subcore_axis_name="s", num_cores=2, num_subcores=16
    )
    fbuf = pltpu.VMEM((_CH, _LANES), jnp.float32)
    return pl.kernel(
        _sc_body,
        out_type=(
            jax.ShapeDtypeStruct((_NW, _CAP), jnp.int32),
            jax.ShapeDtypeStruct((_NW, _CAP), jnp.float32),
            jax.ShapeDtypeStruct((_NW, _CAP), jnp.float32),
            jax.ShapeDtypeStruct((_NW, _CAP), jnp.float32),
            jax.ShapeDtypeStruct((_NW, _CAP), jnp.float32),
            jax.ShapeDtypeStruct((_NW, _CAP), jnp.float32),
        ),
        mesh=mesh,
        compiler_params=pltpu.CompilerParams(needs_layout_passes=False),
        scratch_types=[
            fbuf, fbuf,
            fbuf, fbuf,
            fbuf, fbuf,
            fbuf, fbuf,
            fbuf, fbuf,
            pltpu.VMEM((_CAP,), jnp.float32),
            pltpu.VMEM((_CAP,), jnp.int32),
            pltpu.VMEM((_CAP,), jnp.float32),
            pltpu.VMEM((_CAP,), jnp.float32),
            pltpu.VMEM((_CAP,), jnp.float32),
            pltpu.VMEM((_CAP,), jnp.float32),
            pltpu.SemaphoreType.DMA,
            pltpu.SemaphoreType.DMA,
            pltpu.SemaphoreType.DMA,
            pltpu.SemaphoreType.DMA,
            pltpu.SemaphoreType.DMA,
        ],
    )


def _nms_body(s_ref, x1_ref, y1_ref, x2_ref, y2_ref, lbl_ref,
              ob_ref, os_ref, ol_ref,
              bx1, by1, bx2, by2, bar, basc):
    x1 = x1_ref[...]
    y1 = y1_ref[...]
    x2 = x2_ref[...]
    y2 = y2_ref[...]
    s = s_ref[...]
    lbl = lbl_ref[...]
    act = s > _THR
    neg = jnp.float32(-jnp.inf)
    mc = jnp.max(jnp.where(act, jnp.maximum(jnp.maximum(x1, y1), jnp.maximum(x2, y2)), neg))
    offl = lbl.astype(jnp.float32) * (mc + 1.0)
    ox1 = x1 + offl
    oy1 = y1 + offl
    ox2 = x2 + offl
    oy2 = y2 + offl
    bx1[...] = ox1
    by1[...] = oy1
    bx2[...] = ox2
    by2[...] = oy2
    bar[...] = (ox2 - ox1) * (oy2 - oy1)
    basc[...] = jnp.where(act, s, -1.0)

    lane1 = lax.broadcasted_iota(jnp.int32, (1, 128), 1)
    rio = lax.broadcasted_iota(jnp.int32, (_ROWS, 1), 0)

    def body(t, carry):
        ax1, ay1, ax2, ay2, asv, alv, cnt = carry
        a = basc[...]
        rowmax = jnp.max(a, axis=1, keepdims=True)
        mval = jnp.max(rowmax)
        did = mval > 0.0
        br = jnp.min(jnp.where(rowmax == mval, rio, jnp.int32(_ROWS - 1)))
        arow = basc[pl.ds(br, 1), :]
        bl = jnp.min(jnp.where(arow == mval, lane1, jnp.int32(127)))
        sel = lane1 == bl
        sx1 = jnp.max(jnp.where(sel, x1_ref[pl.ds(br, 1), :], neg))
        sy1 = jnp.max(jnp.where(sel, y1_ref[pl.ds(br, 1), :], neg))
        sx2 = jnp.max(jnp.where(sel, x2_ref[pl.ds(br, 1), :], neg))
        sy2 = jnp.max(jnp.where(sel, y2_ref[pl.ds(br, 1), :], neg))
        slab = jnp.max(jnp.where(sel, lbl_ref[pl.ds(br, 1), :], 0))
        soff = slab.astype(jnp.float32) * (mc + 1.0)
        sbx1 = sx1 + soff
        sby1 = sy1 + soff
        sbx2 = sx2 + soff
        sby2 = sy2 + soff
        sarea = (sbx2 - sbx1) * (sby2 - sby1)
        xx1 = jnp.maximum(sbx1, bx1[...])
        yy1 = jnp.maximum(sby1, by1[...])
        xx2 = jnp.minimum(sbx2, bx2[...])
        yy2 = jnp.minimum(sby2, by2[...])
        inter = jnp.maximum(xx2 - xx1, 0.0) * jnp.maximum(yy2 - yy1, 0.0)
        union = sarea + bar[...] - inter
        iou = inter / jnp.maximum(union, 1e-9)
        sup = iou > _NMS_THR
        basc[...] = jnp.where(jnp.logical_and(did, sup), -1.0, a)
        put = jnp.logical_and(did, lane1 == cnt)
        ax1 = jnp.where(put, sx1, ax1)
        ay1 = jnp.where(put, sy1, ay1)
        ax2 = jnp.where(put, sx2, ax2)
        ay2 = jnp.where(put, sy2, ay2)
        asv = jnp.where(put, mval, asv)
        alv = jnp.where(put, slab, alv)
        cnt = cnt + jnp.where(did, jnp.int32(1), jnp.int32(0))
        return (ax1, ay1, ax2, ay2, asv, alv, cnt)

    z = jnp.zeros((1, 128), jnp.float32)
    zi = jnp.zeros((1, 128), jnp.int32)
    ax1, ay1, ax2, ay2, asv, alv, _ = lax.fori_loop(
        0, _DETS, body, (z, z, z, z, z, zi, jnp.int32(0)))
    ob_ref[...] = jnp.concatenate([ax1, ay1, ax2, ay2], axis=0)
    os_ref[...] = asv
    ol_ref[...] = alv


def _nms(cscore, cx1, cy1, cx2, cy2, clbl):
    return pl.pallas_call(
        _nms_body,
        in_specs=[pl.BlockSpec(memory_space=pltpu.VMEM)] * 6,
        out_shape=(
            jax.ShapeDtypeStruct((4, 128), jnp.float32),
            jax.ShapeDtypeStruct((1, 128), jnp.float32),
            jax.ShapeDtypeStruct((1, 128), jnp.int32),
        ),
        scratch_shapes=[pltpu.VMEM((_ROWS, 128), jnp.float32) for _ in range(6)],
    )(cscore, cx1, cy1, cx2, cy2, clbl)


def kernel(class_logits, box_regression, proposals, image_h, image_w):
    logits_padded = jnp.pad(class_logits, ((0, 0), (0, _LANES - _NCLS)),
                            constant_values=-1e30)
    hw = jnp.stack([jnp.asarray(image_w).astype(jnp.float32),
                    jnp.asarray(image_h).astype(jnp.float32)])
    ms, x1, y1, x2, y2 = _prep(hw, logits_padded, box_regression, proposals)

    clbl, cs, cx1, cy1, cx2, cy2 = _get_sc_kernel()(ms, x1, y1, x2, y2)

    ob, osc, olb = _nms(
        cs.reshape(_ROWS, 128),
        cx1.reshape(_ROWS, 128),
        cy1.reshape(_ROWS, 128),
        cx2.reshape(_ROWS, 128),
        cy2.reshape(_ROWS, 128),
        clbl.reshape(_ROWS, 128),
    )
    out_boxes = ob[:, :_DETS].T
    out_scores = osc[0, :_DETS]
    out_labels = olb[0, :_DETS]
    return (out_boxes, out_scores, out_labels)


# speedup vs baseline: 1.0286x; 1.0286x over previous
import functools
import math

import jax
import jax.numpy as jnp
from jax import lax
from jax.experimental import pallas as pl
from jax.experimental.pallas import tpu as pltpu
from jax.experimental.pallas import tpu_sc as plsc

_N = 20000
_NCLS = 91
_NREG = 364
_LANES = 128
_NW = 32
_SUBCAP = 144
_CAP = 16 * _SUBCAP
_C = _NW * _CAP
_ROWS = _C // 128
_THR = 0.05
_NMS_THR = 0.5
_DETS = 100
_CLIP = math.log(1000.0 / 16)
_RPT = _N // _NW
_SLAB = _RPT + 7
_CH = 80
_NCHUNK = 8


def _prep_body(hw_ref, x_ref, reg_ref, prop_ref,
               ms_ref, x1_ref, y1_ref, x2_ref, y2_ref):
    wf = hw_ref[0]
    hf = hw_ref[1]
    x = x_ref[...]
    mx = jnp.max(x, axis=1, keepdims=True)
    e = jnp.exp(x - mx)
    s = e / jnp.sum(e, axis=1, keepdims=True)

    reg = reg_ref[...]
    i0 = lax.broadcasted_iota(jnp.int32, (_NREG, _LANES), 0)
    i1 = lax.broadcasted_iota(jnp.int32, (_NREG, _LANES), 1)
    s0 = (i0 == i1 * 4).astype(jnp.float32)
    s1 = (i0 == i1 * 4 + 1).astype(jnp.float32)
    s2 = (i0 == i1 * 4 + 2).astype(jnp.float32)
    s3 = (i0 == i1 * 4 + 3).astype(jnp.float32)
    f32 = jnp.float32
    dx = jnp.dot(reg, s0, preferred_element_type=f32) / 10.0
    dy = jnp.dot(reg, s1, preferred_element_type=f32) / 10.0
    dw = jnp.minimum(jnp.dot(reg, s2, preferred_element_type=f32) / 5.0, _CLIP)
    dh = jnp.minimum(jnp.dot(reg, s3, preferred_element_type=f32) / 5.0, _CLIP)

    p = prop_ref[...]
    w = (p[:, 2] - p[:, 0])[:, None]
    h = (p[:, 3] - p[:, 1])[:, None]
    cx = p[:, 0][:, None] + 0.5 * w
    cy = p[:, 1][:, None] + 0.5 * h
    pcx = dx * w + cx
    pcy = dy * h + cy
    pw = jnp.exp(dw) * w
    ph = jnp.exp(dh) * h
    x1 = jnp.clip(pcx - 0.5 * pw, 0.0, wf)
    y1 = jnp.clip(pcy - 0.5 * ph, 0.0, hf)
    x2 = jnp.clip(pcx + 0.5 * pw, 0.0, wf)
    y2 = jnp.clip(pcy + 0.5 * ph, 0.0, hf)

    lane = lax.broadcasted_iota(jnp.int32, x.shape, 1)
    ok = ((lane >= 1) & (lane < _NCLS) & (s > _THR)
          & ((x2 - x1) >= 0.01) & ((y2 - y1) >= 0.01))
    ms_ref[...] = jnp.where(ok, s, 0.0)
    x1_ref[...] = x1
    y1_ref[...] = y1
    x2_ref[...] = x2
    y2_ref[...] = y2


def _prep(hw, logits_padded, box_regression, proposals):
    nb = 10
    br = _N // nb
    out = jax.ShapeDtypeStruct((_N, _LANES), jnp.float32)
    return pl.pallas_call(
        _prep_body,
        grid=(nb,),
        in_specs=[
            pl.BlockSpec(memory_space=pltpu.SMEM),
            pl.BlockSpec((br, _LANES), lambda i: (i, 0)),
            pl.BlockSpec((br, _NREG), lambda i: (i, 0)),
            pl.BlockSpec((br, 4), lambda i: (i, 0)),
        ],
        out_specs=[pl.BlockSpec((br, _LANES), lambda i: (i, 0))] * 5,
        out_shape=(out,) * 5,
    )(hw, logits_padded, box_regression, proposals)


def _sc_body(ms_hbm, x1_hbm, y1_hbm, x2_hbm, y2_hbm,
             clbl_hbm, cs_hbm, cx1_hbm, cy1_hbm, cx2_hbm, cy2_hbm,
             mb0, mb1, xb0, xb1, yb0, yb1, zb0, zb1, wb0, wb1,
             cs, cl, cx1, cy1, cx2, cy2,
             sm, sx, sy, sz, sw):
    wid = lax.axis_index("s") * 2 + lax.axis_index("c")
    rst = pl.multiple_of((wid * _RPT) & -8, 8)
    rlo = wid * _RPT - rst

    iota16 = lax.iota(jnp.int32, 16)
    z16f = jnp.zeros((16,), jnp.float32)

    def zero_body(i, carry):
        cs[pl.ds(i * 16, 16)] = z16f
        return carry

    lax.fori_loop(0, _CAP // 16, zero_body, 0)

    mbufs = (mb0, mb1)
    xbufs = (xb0, xb1)
    ybufs = (yb0, yb1)
    zbufs = (zb0, zb1)
    wbufs = (wb0, wb1)

    def fire(ch):
        b = ch % 2
        r0 = ch * _CH
        nr = _SLAB - r0 if r0 + _CH > _SLAB else _CH
        sl = pl.ds(rst + r0, nr)
        dst = pl.ds(0, nr)
        return (
            pltpu.async_copy(ms_hbm.at[sl], mbufs[b].at[dst], sm),
            pltpu.async_copy(x1_hbm.at[sl], xbufs[b].at[dst], sx),
            pltpu.async_copy(y1_hbm.at[sl], ybufs[b].at[dst], sy),
            pltpu.async_copy(x2_hbm.at[sl], zbufs[b].at[dst], sz),
            pltpu.async_copy(y2_hbm.at[sl], wbufs[b].at[dst], sw),
        )

    cps = fire(0)
    off_v = iota16
    for ch in range(_NCHUNK):
        nxt = fire(ch + 1) if ch + 1 < _NCHUNK else None
        for cp in cps:
            cp.wait()
        cps = nxt
        b = ch % 2
        mbuf = mbufs[b]
        xbuf = xbufs[b]
        ybuf = ybufs[b]
        zbuf = zbufs[b]
        wbuf = wbufs[b]
        r0 = ch * _CH
        nr = _SLAB - r0 if r0 + _CH > _SLAB else _CH
        lo = jnp.maximum(rlo, r0)
        hi = jnp.minimum(rlo + _RPT, r0 + nr)

        def scan_row(lrow, off_v, r0=r0, mbuf=mbuf, xbuf=xbuf, ybuf=ybuf,
                     zbuf=zbuf, wbuf=wbuf):
            crow = lrow - r0
            for lg in range(_LANES // 16):
                slg = pl.ds(lg * 16, 16)
                s = mbuf[crow, slg]
                ok = (s > _THR) & (off_v < _CAP)
                plsc.store_scatter(cs, [off_v], s, mask=ok)
                plsc.store_scatter(cl, [off_v], lg * 16 + iota16, mask=ok)
                plsc.store_scatter(cx1, [off_v], xbuf[crow, slg], mask=ok)
                plsc.store_scatter(cy1, [off_v], ybuf[crow, slg], mask=ok)
                plsc.store_scatter(cx2, [off_v], zbuf[crow, slg], mask=ok)
                plsc.store_scatter(cy2, [off_v], wbuf[crow, slg], mask=ok)
                off_v = off_v + jnp.where(ok, 16, 0)
            return off_v

        off_v = lax.fori_loop(lo, hi, scan_row, off_v)

    pltpu.sync_copy(cl, clbl_hbm.at[wid])
    pltpu.sync_copy(cs, cs_hbm.at[wid])
    pltpu.sync_copy(cx1, cx1_hbm.at[wid])
    pltpu.sync_copy(cy1, cy1_hbm.at[wid])
    pltpu.sync_copy(cx2, cx2_hbm.at[wid])
    pltpu.sync_copy(cy2, cy2_hbm.at[wid])


@functools.cache
def _get_sc_kernel():
    mesh = plsc.VectorSubcoreMesh(
        core_axis_name="c", subcore_axis_name="s", num_cores=2, num_subcores=16
    )
    fbuf = pltpu.VMEM((_CH, _LANES), jnp.float32)
    return pl.kernel(
        _sc_body,
        out_type=(
            jax.ShapeDtypeStruct((_NW, _CAP), jnp.int32),
            jax.ShapeDtypeStruct((_NW, _CAP), jnp.float32),
            jax.ShapeDtypeStruct((_NW, _CAP), jnp.float32),
            jax.ShapeDtypeStruct((_NW, _CAP), jnp.float32),
            jax.ShapeDtypeStruct((_NW, _CAP), jnp.float32),
            jax.ShapeDtypeStruct((_NW, _CAP), jnp.float32),
        ),
        mesh=mesh,
        compiler_params=pltpu.CompilerParams(needs_layout_passes=False),
        scratch_types=[
            fbuf, fbuf,
            fbuf, fbuf,
            fbuf, fbuf,
            fbuf, fbuf,
            fbuf, fbuf,
            pltpu.VMEM((_CAP,), jnp.float32),
            pltpu.VMEM((_CAP,), jnp.int32),
            pltpu.VMEM((_CAP,), jnp.float32),
            pltpu.VMEM((_CAP,), jnp.float32),
            pltpu.VMEM((_CAP,), jnp.float32),
            pltpu.VMEM((_CAP,), jnp.float32),
            pltpu.SemaphoreType.DMA,
            pltpu.SemaphoreType.DMA,
            pltpu.SemaphoreType.DMA,
            pltpu.SemaphoreType.DMA,
            pltpu.SemaphoreType.DMA,
        ],
    )


def _nms_body(s_ref, x1_ref, y1_ref, x2_ref, y2_ref, lbl_ref,
              ob_ref, os_ref, ol_ref,
              bx1, by1, bx2, by2, bar, basc):
    x1 = x1_ref[...]
    y1 = y1_ref[...]
    x2 = x2_ref[...]
    y2 = y2_ref[...]
    s = s_ref[...]
    lbl = lbl_ref[...]
    act = s > _THR
    neg = jnp.float32(-jnp.inf)
    mc = jnp.max(jnp.where(act, jnp.maximum(jnp.maximum(x1, y1), jnp.maximum(x2, y2)), neg))
    offl = lbl.astype(jnp.float32) * (mc + 1.0)
    ox1 = x1 + offl
    oy1 = y1 + offl
    ox2 = x2 + offl
    oy2 = y2 + offl
    bx1[...] = ox1
    by1[...] = oy1
    bx2[...] = ox2
    by2[...] = oy2
    bar[...] = (ox2 - ox1) * (oy2 - oy1)
    basc[...] = jnp.where(act, s, -1.0)

    lin = (lax.broadcasted_iota(jnp.int32, (_ROWS, 128), 0) * 128
           + lax.broadcasted_iota(jnp.int32, (_ROWS, 128), 1))
    lane1 = lax.broadcasted_iota(jnp.int32, (1, 128), 1)

    def body(t, carry):
        ax1, ay1, ax2, ay2, asv, alv, cnt = carry
        a = basc[...]
        mval = jnp.max(a)
        did = mval > 0.0
        bi = jnp.min(jnp.where(a == mval, lin, jnp.int32(_C)))
        bi = jnp.minimum(bi, jnp.int32(_C - 1))
        br = bi >> 7
        bl = bi & 127
        sel = lane1 == bl
        sx1 = jnp.max(jnp.where(sel, x1_ref[pl.ds(br, 1), :], neg))
        sy1 = jnp.max(jnp.where(sel, y1_ref[pl.ds(br, 1), :], neg))
        sx2 = jnp.max(jnp.where(sel, x2_ref[pl.ds(br, 1), :], neg))
        sy2 = jnp.max(jnp.where(sel, y2_ref[pl.ds(br, 1), :], neg))
        slab = jnp.max(jnp.where(sel, lbl_ref[pl.ds(br, 1), :], 0))
        soff = slab.astype(jnp.float32) * (mc + 1.0)
        sbx1 = sx1 + soff
        sby1 = sy1 + soff
        sbx2 = sx2 + soff
        sby2 = sy2 + soff
        sarea = (sbx2 - sbx1) * (sby2 - sby1)
        xx1 = jnp.maximum(sbx1, bx1[...])
        yy1 = jnp.maximum(sby1, by1[...])
        xx2 = jnp.minimum(sbx2, bx2[...])
        yy2 = jnp.minimum(sby2, by2[...])
        inter = jnp.maximum(xx2 - xx1, 0.0) * jnp.maximum(yy2 - yy1, 0.0)
        union = sarea + bar[...] - inter
        iou = inter / jnp.maximum(union, 1e-9)
        sup = iou > _NMS_THR
        basc[...] = jnp.where(jnp.logical_and(did, sup), -1.0, a)
        put = jnp.logical_and(did, lane1 == cnt)
        ax1 = jnp.where(put, sx1, ax1)
        ay1 = jnp.where(put, sy1, ay1)
        ax2 = jnp.where(put, sx2, ax2)
        ay2 = jnp.where(put, sy2, ay2)
        asv = jnp.where(put, mval, asv)
        alv = jnp.where(put, slab, alv)
        cnt = cnt + jnp.where(did, jnp.int32(1), jnp.int32(0))
        return (ax1, ay1, ax2, ay2, asv, alv, cnt)

    z = jnp.zeros((1, 128), jnp.float32)
    zi = jnp.zeros((1, 128), jnp.int32)
    ax1, ay1, ax2, ay2, asv, alv, _ = lax.fori_loop(
        0, _DETS, body, (z, z, z, z, z, zi, jnp.int32(0)))
    ob_ref[...] = jnp.concatenate([ax1, ay1, ax2, ay2], axis=0)
    os_ref[...] = asv
    ol_ref[...] = alv


def _nms(cscore, cx1, cy1, cx2, cy2, clbl):
    return pl.pallas_call(
        _nms_body,
        in_specs=[pl.BlockSpec(memory_space=pltpu.VMEM)] * 6,
        out_shape=(
            jax.ShapeDtypeStruct((4, 128), jnp.float32),
            jax.ShapeDtypeStruct((1, 128), jnp.float32),
            jax.ShapeDtypeStruct((1, 128), jnp.int32),
        ),
        scratch_shapes=[pltpu.VMEM((_ROWS, 128), jnp.float32) for _ in range(6)],
    )(cscore, cx1, cy1, cx2, cy2, clbl)


def kernel(class_logits, box_regression, proposals, image_h, image_w):
    logits_padded = jnp.pad(class_logits, ((0, 0), (0, _LANES - _NCLS)),
                            constant_values=-1e30)
    hw = jnp.stack([jnp.asarray(image_w).astype(jnp.float32),
                    jnp.asarray(image_h).astype(jnp.float32)])
    ms, x1, y1, x2, y2 = _prep(hw, logits_padded, box_regression, proposals)

    clbl, cs, cx1, cy1, cx2, cy2 = _get_sc_kernel()(ms, x1, y1, x2, y2)

    ob, osc, olb = _nms(
        cs.reshape(_ROWS, 128),
        cx1.reshape(_ROWS, 128),
        cy1.reshape(_ROWS, 128),
        cx2.reshape(_ROWS, 128),
        cy2.reshape(_ROWS, 128),
        clbl.reshape(_ROWS, 128),
    )
    out_boxes = ob[:, :_DETS].T
    out_scores = osc[0, :_DETS]
    out_labels = olb[0, :_DETS]
    return (out_boxes, out_scores, out_labels)
